# baseline (speedup 1.0000x reference)
import functools

import jax
import jax.numpy as jnp
from jax import lax
from jax.experimental import pallas as pl
from jax.experimental.pallas import tpu as pltpu
from jax.experimental.pallas import tpu_sc as plsc

_VOCAB = 100000
_D = 128
_B = 4096
_NEG = 20
_NC = 2
_NS = 16
_NW = _NC * _NS
_NB = _B // _NW
_L = 16
_GROUPS = _NB // 8


def _sc_gather_score(center, target, neg_t, wv, wu):
    mesh = plsc.VectorSubcoreMesh(core_axis_name="c", subcore_axis_name="s")

    @functools.partial(
        pl.kernel,
        out_type=[jax.ShapeDtypeStruct((_B // 8, _D), jnp.float32)] * 2,
        mesh=mesh,
        scratch_types=[
            pltpu.VMEM((_NB,), jnp.int32),
            pltpu.VMEM((_NB,), jnp.int32),
            pltpu.VMEM((_NEG, _NB), jnp.int32),
            pltpu.VMEM((_NB, _D), jnp.float32),
            pltpu.VMEM((_NB, _D), jnp.float32),
            pltpu.VMEM((_NB, _D), jnp.float32),
            pltpu.VMEM((_NB, _D), jnp.float32),
            pltpu.VMEM((_GROUPS, _D), jnp.float32),
            pltpu.VMEM((_GROUPS, _D), jnp.float32),
            pltpu.SemaphoreType.DMA,
            pltpu.SemaphoreType.DMA,
            pltpu.SemaphoreType.DMA,
            pltpu.SemaphoreType.DMA,
        ],
    )
    def k(center_hbm, target_hbm, negt_hbm, wv_hbm, wu_hbm,
          pos_out, neg_out,
          cidx, tidx, nidx, cbuf, tbuf, sbufa, sbufb, pdot, ndot,
          sem_c, sem_t, sem_s, sem_n):
        wid = lax.axis_index("s") * _NC + lax.axis_index("c")
        base = wid * _NB
        ni = pltpu.async_copy(negt_hbm.at[:, pl.ds(base, _NB)], nidx, sem_n)
        pltpu.sync_copy(center_hbm.at[pl.ds(base, _NB)], cidx)
        pltpu.sync_copy(target_hbm.at[pl.ds(base, _NB)], tidx)
        cg = pltpu.async_copy(wv_hbm.at[cidx], cbuf, sem_c)
        tg = pltpu.async_copy(wu_hbm.at[tidx], tbuf, sem_t)
        zeros = jnp.zeros((_L,), jnp.float32)

        def zero_body(b, carry):
            for j in range(_D // _L):
                sbufa[b, pl.ds(j * _L, _L)] = zeros
                sbufb[b, pl.ds(j * _L, _L)] = zeros
            return carry

        lax.fori_loop(0, _NB, zero_body, 0)
        ni.wait()
        adds = [
            pltpu.async_copy(wu_hbm.at[nidx.at[kk]],
                             sbufa if kk % 2 == 0 else sbufb, sem_s, add=True)
            for kk in range(_NEG)
        ]
        cg.wait()
        tg.wait()

        def dot_partial(buf_a, buf_b, b):
            acc = buf_a[b, pl.ds(0, _L)] * buf_b[b, pl.ds(0, _L)]
            for j in range(1, _D // _L):
                acc += buf_a[b, pl.ds(j * _L, _L)] * buf_b[b, pl.ds(j * _L, _L)]
            return acc

        def pos_body(i, carry):
            for u in range(2):
                b = 2 * i + u
                pdot[b >> 3, pl.ds((b & 7) * _L, _L)] = dot_partial(
                    cbuf, tbuf, b)
            return carry

        lax.fori_loop(0, _NB // 2, pos_body, 0)
        for a in adds:
            a.wait()

        def neg_partial(b):
            acc = cbuf[b, pl.ds(0, _L)] * (
                sbufa[b, pl.ds(0, _L)] + sbufb[b, pl.ds(0, _L)])
            for j in range(1, _D // _L):
                sl = pl.ds(j * _L, _L)
                acc += cbuf[b, sl] * (sbufa[b, sl] + sbufb[b, sl])
            return acc

        def neg_body(i, carry):
            for u in range(2):
                b = 2 * i + u
                ndot[b >> 3, pl.ds((b & 7) * _L, _L)] = neg_partial(b)
            return carry

        lax.fori_loop(0, _NB // 2, neg_body, 0)
        pltpu.sync_copy(pdot, pos_out.at[pl.ds(wid * _GROUPS, _GROUPS), :])
        pltpu.sync_copy(ndot, neg_out.at[pl.ds(wid * _GROUPS, _GROUPS), :])

    return k(center, target, neg_t, wv, wu)


def _log_sigmoid(x):
    return jnp.minimum(x, 0.0) - jnp.log1p(jnp.exp(-jnp.abs(x)))


def _tc_loss_body(p_ref, n_ref, out_ref):
    col = lax.broadcasted_iota(jnp.int32, (_D, 8), 0)
    grp = lax.broadcasted_iota(jnp.int32, (_D, 8), 1)
    gmat = (col // _L == grp).astype(jnp.float32)
    pos = jnp.dot(p_ref[...], gmat, preferred_element_type=jnp.float32)
    neg = -jnp.dot(n_ref[...], gmat, preferred_element_type=jnp.float32)
    los = _log_sigmoid(pos) + _log_sigmoid(neg)
    out_ref[0, 0] = -jnp.sum(los) / _B


def _tc_loss(pos_part, neg_part):
    out = pl.pallas_call(
        _tc_loss_body,
        out_shape=jax.ShapeDtypeStruct((1, 1), jnp.float32),
        out_specs=pl.BlockSpec(memory_space=pltpu.SMEM),
    )(pos_part, neg_part)
    return out[0, 0]


def kernel(center_words, target_words, negative_words, Wv, Wu):
    center = center_words.reshape(_B).astype(jnp.int32)
    target = target_words.reshape(_B).astype(jnp.int32)
    neg_t = negative_words.astype(jnp.int32).T
    pos_part, neg_part = _sc_gather_score(center, target, neg_t, Wv, Wu)
    return _tc_loss(pos_part, neg_part)


# speedup vs baseline: 1.0127x; 1.0127x over previous
import functools

import jax
import jax.numpy as jnp
from jax import lax
from jax.experimental import pallas as pl
from jax.experimental.pallas import tpu as pltpu
from jax.experimental.pallas import tpu_sc as plsc

_VOCAB = 100000
_D = 128
_B = 4096
_NEG = 20
_NC = 2
_NS = 16
_NW = _NC * _NS
_NB = _B // _NW
_L = 16
_GROUPS = _NB // 8


def _sc_gather_score(center, target, neg_t, wv, wu):
    mesh = plsc.VectorSubcoreMesh(core_axis_name="c", subcore_axis_name="s")

    @functools.partial(
        pl.kernel,
        out_type=[jax.ShapeDtypeStruct((_B // 8, _D), jnp.float32)] * 2,
        mesh=mesh,
        scratch_types=[
            pltpu.VMEM((_NB,), jnp.int32),
            pltpu.VMEM((_NB,), jnp.int32),
            pltpu.VMEM((_NEG, _NB), jnp.int32),
            pltpu.VMEM((_NB, _D), jnp.float32),
            pltpu.VMEM((_NB, _D), jnp.float32),
            pltpu.VMEM((_NB, _D), jnp.float32),
            pltpu.VMEM((_GROUPS, _D), jnp.float32),
            pltpu.VMEM((_GROUPS, _D), jnp.float32),
            pltpu.SemaphoreType.DMA,
            pltpu.SemaphoreType.DMA,
            pltpu.SemaphoreType.DMA,
            pltpu.SemaphoreType.DMA,
        ],
    )
    def k(center_hbm, target_hbm, negt_hbm, wv_hbm, wu_hbm,
          pos_out, neg_out,
          cidx, tidx, nidx, cbuf, tbuf, sbuf, pdot, ndot,
          sem_c, sem_t, sem_s, sem_n):
        wid = lax.axis_index("s") * _NC + lax.axis_index("c")
        base = wid * _NB
        ni = pltpu.async_copy(negt_hbm.at[:, pl.ds(base, _NB)], nidx, sem_n)
        pltpu.sync_copy(center_hbm.at[pl.ds(base, _NB)], cidx)
        pltpu.sync_copy(target_hbm.at[pl.ds(base, _NB)], tidx)
        cg = pltpu.async_copy(wv_hbm.at[cidx], cbuf, sem_c)
        tg = pltpu.async_copy(wu_hbm.at[tidx], tbuf, sem_t)
        zeros = jnp.zeros((_L,), jnp.float32)

        def zero_body(b, carry):
            for j in range(_D // _L):
                sbuf[b, pl.ds(j * _L, _L)] = zeros
            return carry

        lax.fori_loop(0, _NB, zero_body, 0)
        ni.wait()
        adds = [
            pltpu.async_copy(wu_hbm.at[nidx.at[kk]], sbuf, sem_s, add=True)
            for kk in range(_NEG)
        ]
        cg.wait()
        tg.wait()

        def dot_partial(buf_a, buf_b, b):
            acc = buf_a[b, pl.ds(0, _L)] * buf_b[b, pl.ds(0, _L)]
            for j in range(1, _D // _L):
                acc += buf_a[b, pl.ds(j * _L, _L)] * buf_b[b, pl.ds(j * _L, _L)]
            return acc

        def pos_body(i, carry):
            for u in range(2):
                b = 2 * i + u
                pdot[b >> 3, pl.ds((b & 7) * _L, _L)] = dot_partial(
                    cbuf, tbuf, b)
            return carry

        lax.fori_loop(0, _NB // 2, pos_body, 0)
        for a in adds:
            a.wait()

        def neg_body(i, carry):
            for u in range(2):
                b = 2 * i + u
                ndot[b >> 3, pl.ds((b & 7) * _L, _L)] = dot_partial(
                    cbuf, sbuf, b)
            return carry

        lax.fori_loop(0, _NB // 2, neg_body, 0)
        pltpu.sync_copy(pdot, pos_out.at[pl.ds(wid * _GROUPS, _GROUPS), :])
        pltpu.sync_copy(ndot, neg_out.at[pl.ds(wid * _GROUPS, _GROUPS), :])

    return k(center, target, neg_t, wv, wu)


def _log_sigmoid(x):
    return jnp.minimum(x, 0.0) - jnp.log1p(jnp.exp(-jnp.abs(x)))


def _tc_loss_body(p_ref, n_ref, out_ref):
    col = lax.broadcasted_iota(jnp.int32, (_D, 8), 0)
    grp = lax.broadcasted_iota(jnp.int32, (_D, 8), 1)
    gmat = (col // _L == grp).astype(jnp.float32)
    pos = jnp.dot(p_ref[...], gmat, preferred_element_type=jnp.float32)
    neg = -jnp.dot(n_ref[...], gmat, preferred_element_type=jnp.float32)
    los = _log_sigmoid(pos) + _log_sigmoid(neg)
    out_ref[0, 0] = -jnp.sum(los) / _B


def _tc_loss(pos_part, neg_part):
    out = pl.pallas_call(
        _tc_loss_body,
        out_shape=jax.ShapeDtypeStruct((1, 1), jnp.float32),
        out_specs=pl.BlockSpec(memory_space=pltpu.SMEM),
    )(pos_part, neg_part)
    return out[0, 0]


def kernel(center_words, target_words, negative_words, Wv, Wu):
    center = center_words.reshape(_B).astype(jnp.int32)
    target = target_words.reshape(_B).astype(jnp.int32)
    neg_t = negative_words.astype(jnp.int32).T
    pos_part, neg_part = _sc_gather_score(center, target, neg_t, Wv, Wu)
    return _tc_loss(pos_part, neg_part)


# speedup vs baseline: 1.0193x; 1.0065x over previous
import functools

import jax
import jax.numpy as jnp
from jax import lax
from jax.experimental import pallas as pl
from jax.experimental.pallas import tpu as pltpu
from jax.experimental.pallas import tpu_sc as plsc

_VOCAB = 100000
_D = 128
_B = 4096
_NEG = 20
_NC = 2
_NS = 16
_NW = _NC * _NS
_NB = _B // _NW
_L = 16
_GROUPS = _NB // 8


def _sc_gather_score(center, target, neg_t, wv, wu):
    mesh = plsc.VectorSubcoreMesh(core_axis_name="c", subcore_axis_name="s")

    @functools.partial(
        pl.kernel,
        out_type=[jax.ShapeDtypeStruct((_B // 8, _D), jnp.float32)] * 2,
        mesh=mesh,
        scratch_types=[
            pltpu.VMEM((_NB,), jnp.int32),
            pltpu.VMEM((_NB,), jnp.int32),
            pltpu.VMEM((_NEG, _NB), jnp.int32),
            pltpu.VMEM((_NB, _D), jnp.float32),
            pltpu.VMEM((_NB, _D), jnp.float32),
            pltpu.VMEM((_NB, _D), jnp.float32),
            pltpu.VMEM((_GROUPS, _D), jnp.float32),
            pltpu.VMEM((_GROUPS, _D), jnp.float32),
            pltpu.SemaphoreType.DMA,
            pltpu.SemaphoreType.DMA,
            pltpu.SemaphoreType.DMA,
            pltpu.SemaphoreType.DMA,
        ],
    )
    def k(center_hbm, target_hbm, negt_hbm, wv_hbm, wu_hbm,
          pos_out, neg_out,
          cidx, tidx, nidx, cbuf, tbuf, sbuf, pdot, ndot,
          sem_c, sem_t, sem_s, sem_n):
        wid = lax.axis_index("s") * _NC + lax.axis_index("c")
        base = wid * _NB
        ni = pltpu.async_copy(negt_hbm.at[:, pl.ds(base, _NB)], nidx, sem_n)
        pltpu.sync_copy(center_hbm.at[pl.ds(base, _NB)], cidx)
        pltpu.sync_copy(target_hbm.at[pl.ds(base, _NB)], tidx)
        cg = pltpu.async_copy(wv_hbm.at[cidx], cbuf, sem_c)
        tg = pltpu.async_copy(wu_hbm.at[tidx], tbuf, sem_t)
        zeros = jnp.zeros((_L,), jnp.float32)

        def zero_body(b, carry):
            for j in range(_D // _L):
                sbuf[b, pl.ds(j * _L, _L)] = zeros
            return carry

        lax.fori_loop(0, _NB, zero_body, 0)
        ni.wait()
        _H = _NB // 2
        adds_lo = [
            pltpu.async_copy(wu_hbm.at[nidx.at[kk, pl.ds(0, _H)]],
                             sbuf.at[pl.ds(0, _H), :], sem_s, add=True)
            for kk in range(_NEG)
        ]
        adds_hi = [
            pltpu.async_copy(wu_hbm.at[nidx.at[kk, pl.ds(_H, _H)]],
                             sbuf.at[pl.ds(_H, _H), :], sem_n, add=True)
            for kk in range(_NEG)
        ]
        cg.wait()
        tg.wait()

        def dot_partial(buf_a, buf_b, b):
            acc = buf_a[b, pl.ds(0, _L)] * buf_b[b, pl.ds(0, _L)]
            for j in range(1, _D // _L):
                acc += buf_a[b, pl.ds(j * _L, _L)] * buf_b[b, pl.ds(j * _L, _L)]
            return acc

        def pos_body(i, carry):
            for u in range(2):
                b = 2 * i + u
                pdot[b >> 3, pl.ds((b & 7) * _L, _L)] = dot_partial(
                    cbuf, tbuf, b)
            return carry

        lax.fori_loop(0, _NB // 2, pos_body, 0)

        def neg_body(i, carry):
            for u in range(2):
                b = 2 * i + u
                ndot[b >> 3, pl.ds((b & 7) * _L, _L)] = dot_partial(
                    cbuf, sbuf, b)
            return carry

        for a in adds_lo:
            a.wait()
        lax.fori_loop(0, _H // 2, neg_body, 0)
        for a in adds_hi:
            a.wait()
        lax.fori_loop(_H // 2, _NB // 2, neg_body, 0)
        pltpu.sync_copy(pdot, pos_out.at[pl.ds(wid * _GROUPS, _GROUPS), :])
        pltpu.sync_copy(ndot, neg_out.at[pl.ds(wid * _GROUPS, _GROUPS), :])

    return k(center, target, neg_t, wv, wu)


def _log_sigmoid(x):
    return jnp.minimum(x, 0.0) - jnp.log1p(jnp.exp(-jnp.abs(x)))


def _tc_loss_body(p_ref, n_ref, out_ref):
    col = lax.broadcasted_iota(jnp.int32, (_D, 8), 0)
    grp = lax.broadcasted_iota(jnp.int32, (_D, 8), 1)
    gmat = (col // _L == grp).astype(jnp.float32)
    pos = jnp.dot(p_ref[...], gmat, preferred_element_type=jnp.float32)
    neg = -jnp.dot(n_ref[...], gmat, preferred_element_type=jnp.float32)
    los = _log_sigmoid(pos) + _log_sigmoid(neg)
    out_ref[0, 0] = -jnp.sum(los) / _B


def _tc_loss(pos_part, neg_part):
    out = pl.pallas_call(
        _tc_loss_body,
        out_shape=jax.ShapeDtypeStruct((1, 1), jnp.float32),
        out_specs=pl.BlockSpec(memory_space=pltpu.SMEM),
    )(pos_part, neg_part)
    return out[0, 0]


def kernel(center_words, target_words, negative_words, Wv, Wu):
    center = center_words.reshape(_B).astype(jnp.int32)
    target = target_words.reshape(_B).astype(jnp.int32)
    neg_t = negative_words.astype(jnp.int32).T
    pos_part, neg_part = _sc_gather_score(center, target, neg_t, Wv, Wu)
    return _tc_loss(pos_part, neg_part)
